# baseline (speedup 1.0000x reference)
import functools
import jax
import jax.numpy as jnp
from jax import lax
from jax.experimental import pallas as pl
from jax.experimental.pallas import tpu as pltpu
from jax.experimental.pallas import tpu_sc as plsc

_B = 4
_S = 4096
_D = 1024
_NW = 32
_ROWS_W = _S // _NW
_CS = 8
_NCH = _ROWS_W // _CS


def _sc_add(seq_hbm, pe_hbm, out_hbm,
            s0_buf, s1_buf, s2_buf, s3_buf,
            o0_buf, o1_buf, o2_buf, o3_buf,
            pe0_buf, pe1_buf,
            i0_sem, i1_sem, i2_sem, i3_sem,
            u0_sem, u1_sem, u2_sem, u3_sem,
            p0_sem, p1_sem):
    seq_bufs = (s0_buf, s1_buf, s2_buf, s3_buf)
    out_bufs = (o0_buf, o1_buf, o2_buf, o3_buf)
    pe_bufs = (pe0_buf, pe1_buf)
    in_sems = (i0_sem, i1_sem, i2_sem, i3_sem)
    out_sems = (u0_sem, u1_sem, u2_sem, u3_sem)
    pe_sems = (p0_sem, p1_sem)

    wid = lax.axis_index("s") * 2 + lax.axis_index("c")
    s_base = wid * _ROWS_W

    def in_copy(c, b):
        return pltpu.make_async_copy(
            seq_hbm.at[b, pl.ds(s_base + c * _CS, _CS), :],
            seq_bufs[b], in_sems[b])

    def out_copy(c, b):
        return pltpu.make_async_copy(
            out_bufs[b],
            out_hbm.at[b, pl.ds(s_base + c * _CS, _CS), :],
            out_sems[b])

    def pe_copy(c, p):
        return pltpu.make_async_copy(
            pe_hbm.at[pl.ds(s_base + c * _CS, _CS), :],
            pe_bufs[p], pe_sems[p])

    pe_copy(0, 0).start()
    pe_copy(1, 1).start()
    for b in range(_B):
        in_copy(0, b).start()

    def do_chunk(c, parity, is_first, has_next2):
        pe_buf = pe_bufs[parity]
        pe_copy(c, parity).wait()

        def row_add(r, carry):
            for j in range(_D // 16):
                sl = pl.ds(j * 16, 16)
                v_pe = pe_buf[r, sl]
                for b in range(_B):
                    out_bufs[b][r, sl] = seq_bufs[b][r, sl] + v_pe
            return carry

        for b in range(_B):
            in_copy(c, b).wait()

            if isinstance(is_first, bool):
                if not is_first:
                    out_copy(c - 1, b).wait()
            else:
                @pl.when(jnp.logical_not(is_first))
                def _():
                    out_copy(c - 1, b).wait()

        lax.fori_loop(0, _CS, row_add, 0)

        for b in range(_B):
            out_copy(c, b).start()

            @pl.when(c + 1 <= _NCH - 1)
            def _():
                in_copy(c + 1, b).start()

        @pl.when(has_next2)
        def _():
            pe_copy(c + 2, parity).start()

    def body(cc, carry):
        c0 = 2 * cc
        do_chunk(c0, 0, cc == 0, cc < _NCH // 2 - 1)
        do_chunk(c0 + 1, 1, False, cc < _NCH // 2 - 1)
        return carry

    lax.fori_loop(0, _NCH // 2, body, 0)

    for b in range(_B):
        out_copy(_NCH - 1, b).wait()


def kernel(sequence, pe):
    batch, seq_len, d_model = sequence.shape
    pe2d = pe[0, :seq_len]

    mesh = plsc.VectorSubcoreMesh(core_axis_name="c", subcore_axis_name="s")
    f32 = jnp.float32
    run = functools.partial(
        pl.kernel, mesh=mesh,
        out_type=jax.ShapeDtypeStruct((batch, seq_len, d_model), f32),
        scratch_types=(
            [pltpu.VMEM((_CS, _D), f32)] * 8
            + [pltpu.VMEM((_CS, _D), f32)] * 2
            + [pltpu.SemaphoreType.DMA] * 10
        ),
    )(_sc_add)
    return run(sequence, pe2d)


# speedup vs baseline: 1.2434x; 1.2434x over previous
import functools
import jax
import jax.numpy as jnp
from jax import lax
from jax.experimental import pallas as pl
from jax.experimental.pallas import tpu as pltpu
from jax.experimental.pallas import tpu_sc as plsc

_B = 4
_S = 4096
_D = 1024

_S1 = 2816
_S2 = _S - _S1
_TC_BLOCK = 256

_NW = 32
_RW = _S2 // _NW
_CS = 8
_NCH = _RW // _CS


def _sc_add(seq_hbm, pe_hbm, out_hbm,
            s0_buf, s1_buf, s2_buf, s3_buf,
            o0_buf, o1_buf, o2_buf, o3_buf,
            pe0_buf, pe1_buf,
            i0_sem, i1_sem, i2_sem, i3_sem,
            u0_sem, u1_sem, u2_sem, u3_sem,
            p0_sem, p1_sem):
    seq_bufs = (s0_buf, s1_buf, s2_buf, s3_buf)
    out_bufs = (o0_buf, o1_buf, o2_buf, o3_buf)
    pe_bufs = (pe0_buf, pe1_buf)
    in_sems = (i0_sem, i1_sem, i2_sem, i3_sem)
    out_sems = (u0_sem, u1_sem, u2_sem, u3_sem)
    pe_sems = (p0_sem, p1_sem)

    wid = lax.axis_index("s") * 2 + lax.axis_index("c")
    local_base = wid * _RW
    hbm_base = _S1 + local_base

    def in_copy(c, b):
        return pltpu.make_async_copy(
            seq_hbm.at[b, pl.ds(hbm_base + c * _CS, _CS), :],
            seq_bufs[b], in_sems[b])

    def out_copy(c, b):
        return pltpu.make_async_copy(
            out_bufs[b],
            out_hbm.at[b, pl.ds(local_base + c * _CS, _CS), :],
            out_sems[b])

    def pe_copy(c, p):
        return pltpu.make_async_copy(
            pe_hbm.at[pl.ds(hbm_base + c * _CS, _CS), :],
            pe_bufs[p], pe_sems[p])

    pe_copy(0, 0).start()
    if _NCH > 1:
        pe_copy(1, 1).start()
    for b in range(_B):
        in_copy(0, b).start()

    for c in range(_NCH):
        parity = c % 2
        pe_buf = pe_bufs[parity]
        pe_copy(c, parity).wait()

        for b in range(_B):
            in_copy(c, b).wait()
            if c > 0:
                out_copy(c - 1, b).wait()

        def row_add(r, carry, pe_buf=pe_buf):
            for j in range(_D // 16):
                sl = pl.ds(j * 16, 16)
                v_pe = pe_buf[r, sl]
                for b in range(_B):
                    out_bufs[b][r, sl] = seq_bufs[b][r, sl] + v_pe
            return carry

        lax.fori_loop(0, _CS, row_add, 0)

        for b in range(_B):
            out_copy(c, b).start()
            if c + 1 < _NCH:
                in_copy(c + 1, b).start()
        if c + 2 < _NCH:
            pe_copy(c + 2, parity).start()

    for b in range(_B):
        out_copy(_NCH - 1, b).wait()


def _tc_add(seq_ref, pe_ref, out_ref):
    out_ref[...] = seq_ref[...] + pe_ref[...][None, :, :]


def _tc_merge(full_ref, src_ref, out_ref):
    out_ref[...] = src_ref[...]


def kernel(sequence, pe):
    batch, seq_len, d_model = sequence.shape
    pe2d = pe[0, :seq_len]
    f32 = jnp.float32

    mesh = plsc.VectorSubcoreMesh(core_axis_name="c", subcore_axis_name="s")
    sc_run = functools.partial(
        pl.kernel, mesh=mesh,
        out_type=jax.ShapeDtypeStruct((batch, _S2, d_model), f32),
        scratch_types=(
            [pltpu.VMEM((_CS, _D), f32)] * 10
            + [pltpu.SemaphoreType.DMA] * 10
        ),
    )(_sc_add)
    sc_part = sc_run(sequence, pe2d)

    tc_out = pl.pallas_call(
        _tc_add,
        grid=(_S1 // _TC_BLOCK,),
        in_specs=[
            pl.BlockSpec((batch, _TC_BLOCK, d_model), lambda s: (0, s, 0)),
            pl.BlockSpec((_TC_BLOCK, d_model), lambda s: (s, 0)),
        ],
        out_specs=pl.BlockSpec((batch, _TC_BLOCK, d_model),
                               lambda s: (0, s, 0)),
        out_shape=jax.ShapeDtypeStruct(sequence.shape, sequence.dtype),
    )(sequence, pe2d)

    out = pl.pallas_call(
        _tc_merge,
        grid=(_S2 // _TC_BLOCK,),
        in_specs=[
            pl.BlockSpec(memory_space=pl.ANY),
            pl.BlockSpec((batch, _TC_BLOCK, d_model), lambda s: (0, s, 0)),
        ],
        out_specs=pl.BlockSpec((batch, _TC_BLOCK, d_model),
                               lambda s: (0, s + _S1 // _TC_BLOCK, 0)),
        out_shape=jax.ShapeDtypeStruct(sequence.shape, sequence.dtype),
        input_output_aliases={0: 0},
    )(tc_out, sc_part)
    return out


# speedup vs baseline: 1.2517x; 1.0066x over previous
import functools
import jax
import jax.numpy as jnp
from jax import lax
from jax.experimental import pallas as pl
from jax.experimental.pallas import tpu as pltpu
from jax.experimental.pallas import tpu_sc as plsc

_B = 4
_S = 4096
_D = 1024

_S1 = 2816
_S2 = _S - _S1
_TC_BLOCK = 256

_NW = 32
_RW = _S2 // _NW
_CS = 8
_NCH = _RW // _CS


def _sc_add(seq_hbm, pe_hbm, out_hbm,
            s0_buf, s1_buf, s2_buf, s3_buf,
            o0_buf, o1_buf, o2_buf, o3_buf,
            pe0_buf, pe1_buf,
            i0_sem, i1_sem, i2_sem, i3_sem,
            u0_sem, u1_sem, u2_sem, u3_sem,
            p0_sem, p1_sem):
    seq_bufs = (s0_buf, s1_buf, s2_buf, s3_buf)
    out_bufs = (o0_buf, o1_buf, o2_buf, o3_buf)
    pe_bufs = (pe0_buf, pe1_buf)
    in_sems = (i0_sem, i1_sem, i2_sem, i3_sem)
    out_sems = (u0_sem, u1_sem, u2_sem, u3_sem)
    pe_sems = (p0_sem, p1_sem)

    wid = lax.axis_index("s") * 2 + lax.axis_index("c")
    local_base = wid * _RW
    hbm_base = _S1 + local_base

    def in_copy(c, b):
        return pltpu.make_async_copy(
            seq_hbm.at[b, pl.ds(hbm_base + c * _CS, _CS), :],
            seq_bufs[b], in_sems[b])

    def out_copy(c, b):
        return pltpu.make_async_copy(
            out_bufs[b],
            out_hbm.at[b, pl.ds(local_base + c * _CS, _CS), :],
            out_sems[b])

    def pe_copy(c, p):
        return pltpu.make_async_copy(
            pe_hbm.at[pl.ds(hbm_base + c * _CS, _CS), :],
            pe_bufs[p], pe_sems[p])

    pe_copy(0, 0).start()
    if _NCH > 1:
        pe_copy(1, 1).start()
    for b in range(_B):
        in_copy(0, b).start()

    for c in range(_NCH):
        parity = c % 2
        pe_buf = pe_bufs[parity]
        pe_copy(c, parity).wait()

        for b in range(_B):
            in_copy(c, b).wait()
            if c > 0:
                out_copy(c - 1, b).wait()

        def row_add(r, carry, pe_buf=pe_buf):
            for j in range(_D // 16):
                sl = pl.ds(j * 16, 16)
                v_pe = pe_buf[r, sl]
                for b in range(_B):
                    out_bufs[b][r, sl] = seq_bufs[b][r, sl] + v_pe
            return carry

        lax.fori_loop(0, _CS, row_add, 0)

        for b in range(_B):
            out_copy(c, b).start()
            if c + 1 < _NCH:
                in_copy(c + 1, b).start()
        if c + 2 < _NCH:
            pe_copy(c + 2, parity).start()

    for b in range(_B):
        out_copy(_NCH - 1, b).wait()


def _tc_add(seq_ref, pe_ref, out_ref):
    out_ref[...] = seq_ref[...] + pe_ref[...][None, :, :]


def _tc_merge(full_ref, src_ref, out_ref):
    out_ref[...] = src_ref[...]


def kernel(sequence, pe):
    batch, seq_len, d_model = sequence.shape
    pe2d = pe[0, :seq_len]
    f32 = jnp.float32

    mesh = plsc.VectorSubcoreMesh(core_axis_name="c", subcore_axis_name="s")
    sc_run = functools.partial(
        pl.kernel, mesh=mesh,
        out_type=jax.ShapeDtypeStruct((batch, _S2, d_model), f32),
        scratch_types=(
            [pltpu.VMEM((_CS, _D), f32)] * 10
            + [pltpu.SemaphoreType.DMA] * 10
        ),
    )(_sc_add)
    tc_out = pl.pallas_call(
        _tc_add,
        grid=(_S1 // _TC_BLOCK,),
        in_specs=[
            pl.BlockSpec((batch, _TC_BLOCK, d_model), lambda s: (0, s, 0)),
            pl.BlockSpec((_TC_BLOCK, d_model), lambda s: (s, 0)),
        ],
        out_specs=pl.BlockSpec((batch, _TC_BLOCK, d_model),
                               lambda s: (0, s, 0)),
        out_shape=jax.ShapeDtypeStruct(sequence.shape, sequence.dtype),
    )(sequence, pe2d)

    sc_part = sc_run(sequence, pe2d)

    out = pl.pallas_call(
        _tc_merge,
        grid=(_S2 // _TC_BLOCK,),
        in_specs=[
            pl.BlockSpec(memory_space=pl.ANY),
            pl.BlockSpec((batch, _TC_BLOCK, d_model), lambda s: (0, s, 0)),
        ],
        out_specs=pl.BlockSpec((batch, _TC_BLOCK, d_model),
                               lambda s: (0, s + _S1 // _TC_BLOCK, 0)),
        out_shape=jax.ShapeDtypeStruct(sequence.shape, sequence.dtype),
        input_output_aliases={0: 0},
    )(tc_out, sc_part)
    return out


# speedup vs baseline: 2.2061x; 1.7625x over previous
import jax
import jax.numpy as jnp
from jax.experimental import pallas as pl
from jax.experimental.pallas import tpu as pltpu

_CHUNK_S = 512
_NBUF = 4


def _pipeline_kernel(seq_hbm, pe_hbm, out_hbm,
                     seq_buf, pe_buf, seq_sem, pe_sem, out_sem):
    batch, seq_len, d_model = seq_hbm.shape
    nchunk = seq_len // _CHUNK_S

    def seq_copy(i, slot):
        return pltpu.make_async_copy(
            seq_hbm.at[:, pl.ds(i * _CHUNK_S, _CHUNK_S), :],
            seq_buf.at[slot], seq_sem.at[slot])

    def pe_copy(i, slot):
        return pltpu.make_async_copy(
            pe_hbm.at[pl.ds(i * _CHUNK_S, _CHUNK_S), :],
            pe_buf.at[slot], pe_sem.at[slot])

    def out_copy(i, slot):
        return pltpu.make_async_copy(
            seq_buf.at[slot],
            out_hbm.at[:, pl.ds(i * _CHUNK_S, _CHUNK_S), :],
            out_sem.at[slot])

    for i in range(min(_NBUF, nchunk)):
        seq_copy(i, i).start()
        pe_copy(i, i).start()

    for i in range(nchunk):
        slot = i % _NBUF
        seq_copy(i, slot).wait()
        pe_copy(i, slot).wait()
        seq_buf[slot] = seq_buf[slot] + pe_buf[slot][None, :, :]
        out_copy(i, slot).start()
        nxt = i + _NBUF
        if nxt < nchunk:
            out_copy(i, slot).wait()
            seq_copy(nxt, slot).start()
            pe_copy(nxt, slot).start()

    for i in range(max(nchunk - _NBUF, 0), nchunk):
        out_copy(i, i % _NBUF).wait()


def kernel(sequence, pe):
    batch, seq_len, d_model = sequence.shape
    pe2d = pe[0, :seq_len]

    out = pl.pallas_call(
        _pipeline_kernel,
        in_specs=[
            pl.BlockSpec(memory_space=pl.ANY),
            pl.BlockSpec(memory_space=pl.ANY),
        ],
        out_specs=pl.BlockSpec(memory_space=pl.ANY),
        out_shape=jax.ShapeDtypeStruct(sequence.shape, sequence.dtype),
        scratch_shapes=[
            pltpu.VMEM((_NBUF, batch, _CHUNK_S, d_model), jnp.float32),
            pltpu.VMEM((_NBUF, _CHUNK_S, d_model), jnp.float32),
            pltpu.SemaphoreType.DMA((_NBUF,)),
            pltpu.SemaphoreType.DMA((_NBUF,)),
            pltpu.SemaphoreType.DMA((_NBUF,)),
        ],
    )(sequence, pe2d)
    return out
